# baseline (speedup 1.0000x reference)
import functools

import jax
import jax.numpy as jnp
from jax import lax
from jax.experimental import pallas as pl
from jax.experimental.pallas import tpu as pltpu
from jax.experimental.pallas import tpu_sc as plsc

N = 10000
E = 320000
D = 128
R = 8
NC = 2
NS = 16
C = 5000
WPC = 8
EPT = E // NS
SEG = 4000
NSEG = EPT // SEG
MCAP = SEG + 256
CH = 256
WROWS = 16 * 320


def _make_bucket_body(with_deg):
    def _bucket_body(*args):
        if with_deg:
            (x_hbm, src_hbm, comb_hbm, bucket_hbm, deg_hbm,
             seg_src0, seg_comb0, seg_src1, seg_comb1, m_src, m_comb,
             stag0, stag1, zeros1d, ones1d,
             idx0, sidx0, idx1, sidx1, deg_st, esem, gsem, ssem,
             win, degwin) = args
        else:
            (x_hbm, src_hbm, comb_hbm, bucket_hbm,
             seg_src0, seg_comb0, seg_src1, seg_comb1, m_src, m_comb,
             stag0, stag1, zeros1d, ones1d,
             idx0, sidx0, idx1, sidx1, deg_st, esem, gsem, ssem,
             win, degwin) = args
            deg_hbm = None
        c = lax.axis_index("c")
        s = lax.axis_index("s")

        zf = jnp.zeros((16,), jnp.float32)
        of = jnp.ones((16,), jnp.float32)
        zi = jnp.zeros((16,), jnp.int32)
        cv = jnp.full((16,), C, jnp.int32)
        iota = lax.iota(jnp.int32, 16)

        segs = ((seg_src0, seg_comb0), (seg_src1, seg_comb1))
        stags = (stag0, stag1)
        idxs = ((idx0, sidx0), (idx1, sidx1))

        for u in range(32):
            zeros1d[pl.ds(u * 16, 16)] = zf
        for u in range(16):
            ones1d[pl.ds(u * 16, 16)] = of

        ebase = s * EPT

        def _load_seg(g):
            eoff = pl.multiple_of(ebase + g * SEG, 8)
            pltpu.async_copy(src_hbm.at[pl.ds(eoff, SEG)], seg_src0, esem)
            pltpu.async_copy(comb_hbm.at[pl.ds(eoff, SEG)], seg_comb0, esem)
            pltpu.make_async_copy(src_hbm.at[pl.ds(0, SEG)], seg_src0, esem).wait()
            pltpu.make_async_copy(comb_hbm.at[pl.ds(0, SEG)], seg_comb0, esem).wait()

        def _window(w, _):
            wid = c * WPC + w
            base = wid * C
            r0 = s * 320

            def _zs(i, _):
                for u in range(8):
                    stag0[i, pl.ds(u * 16, 16)] = zf
                return 0
            lax.fori_loop(0, 128, _zs, 0)
            pltpu.sync_copy(stag0.at[pl.ds(0, 128)], win.at[pl.ds(r0, 128)])
            pltpu.sync_copy(stag0.at[pl.ds(0, 128)], win.at[pl.ds(r0 + 128, 128)])
            pltpu.sync_copy(stag0.at[pl.ds(0, 64)], win.at[pl.ds(r0 + 256, 64)])
            if with_deg:
                pltpu.sync_copy(zeros1d.at[pl.ds(0, 320)],
                                degwin.at[pl.ds(r0, 320)])
            plsc.subcore_barrier()

            def _process_seg(g, p):
                seg_srcp, seg_combp = segs[p]

                def _scan(i, cntv):
                    vc = seg_combp[pl.ds(i * 16, 16)]
                    vs = seg_srcp[pl.ds(i * 16, 16)]
                    rel = vc - base
                    msk = (rel >= 0) & (rel < C)
                    mi = msk.astype(jnp.int32)
                    pfx = plsc.cumsum(mi)
                    dest = cntv + pfx - 1
                    plsc.store_scatter(m_comb, [dest], rel, mask=msk)
                    plsc.store_scatter(m_src, [dest], vs, mask=msk)
                    return cntv + plsc.all_reduce_population_count(msk)
                cntv = lax.fori_loop(0, SEG // 16, _scan,
                                     jnp.zeros((16,), jnp.int32))

                for u in range(16):
                    pdest = cntv + iota + u * 16
                    plsc.store_scatter(m_comb, [pdest], cv)
                    plsc.store_scatter(m_src, [pdest], zi)

                cnt0 = cntv[0]
                nch = (cnt0 + (CH - 1)) // CH

                def _build_idx(j, q):
                    for u in range(16):
                        gidx = (jnp.full((16,), 0, jnp.int32)
                                + j * CH + u * 16 + iota)
                        idxs[q][0][pl.ds(u * 16, 16)] = plsc.load_gather(
                            m_comb, [gidx])
                        idxs[q][1][pl.ds(u * 16, 16)] = plsc.load_gather(
                            m_src, [gidx])

                def _wait_sca(q):
                    pltpu.make_async_copy(stags[q], win.at[idxs[q][0]],
                                          ssem).wait()
                    if with_deg:
                        pltpu.make_async_copy(ones1d, degwin.at[idxs[q][0]],
                                              ssem).wait()

                @pl.when(nch > 0)
                def _():
                    _build_idx(0, 0)
                    pltpu.async_copy(x_hbm.at[idxs[0][1]], stags[0], gsem)

                def _chunk(j, _):
                    for q in (0, 1):
                        @pl.when((j & 1) == q)
                        def _():
                            pltpu.make_async_copy(x_hbm.at[idxs[q][1]],
                                                  stags[q], gsem).wait()
                            @pl.when(j + 1 < nch)
                            def _():
                                @pl.when(j >= 1)
                                def _():
                                    _wait_sca(1 - q)
                                _build_idx(j + 1, 1 - q)
                                pltpu.async_copy(x_hbm.at[idxs[1 - q][1]],
                                                 stags[1 - q], gsem)
                            pltpu.async_copy(stags[q], win.at[idxs[q][0]],
                                             ssem, add=True)
                            if with_deg:
                                pltpu.async_copy(ones1d,
                                                 degwin.at[idxs[q][0]],
                                                 ssem, add=True)
                    return 0
                lax.fori_loop(0, nch, _chunk, 0)
                for q in (0, 1):
                    @pl.when((nch > 1) & (((nch - 2) & 1) == q))
                    def _():
                        _wait_sca(q)
                    @pl.when((nch > 0) & (((nch - 1) & 1) == q))
                    def _():
                        _wait_sca(q)

            def _segment(g, _):
                _load_seg(g)
                _process_seg(g, 0)
                return 0
            lax.fori_loop(0, NSEG, _segment, 0)

            plsc.subcore_barrier()

            @pl.when(s < 15)
            def _():
                pltpu.sync_copy(win.at[pl.ds(r0, 320)],
                                bucket_hbm.at[pl.ds(base + r0, 320)])
                if with_deg:
                    pltpu.sync_copy(degwin.at[pl.ds(r0, 320)],
                                    deg_st.at[pl.ds(0, 320)])
                    pltpu.sync_copy(deg_st.at[pl.ds(0, 320)],
                                    deg_hbm.at[pl.ds(base + r0, 320)])
            @pl.when(s == 15)
            def _():
                pltpu.sync_copy(win.at[pl.ds(15 * 320, 200)],
                                bucket_hbm.at[pl.ds(base + 15 * 320, 200)])
                if with_deg:
                    pltpu.sync_copy(degwin.at[pl.ds(15 * 320, 200)],
                                    deg_st.at[pl.ds(0, 200)])
                    pltpu.sync_copy(deg_st.at[pl.ds(0, 200)],
                                    deg_hbm.at[pl.ds(base + 15 * 320, 200)])
            return 0

        lax.fori_loop(0, WPC, _window, 0)
    return _bucket_body


def _bucket(x, src, comb, with_deg):
    mesh = plsc.VectorSubcoreMesh(core_axis_name="c", subcore_axis_name="s")
    if with_deg:
        out_type = (jax.ShapeDtypeStruct((N * R, D), jnp.float32),
                    jax.ShapeDtypeStruct((N * R,), jnp.float32))
    else:
        out_type = jax.ShapeDtypeStruct((N * R, D), jnp.float32)
    f = pl.kernel(
        _make_bucket_body(with_deg),
        out_type=out_type,
        mesh=mesh,
        scratch_types=[
            pltpu.VMEM((SEG,), jnp.int32),
            pltpu.VMEM((SEG,), jnp.int32),
            pltpu.VMEM((8,), jnp.int32),
            pltpu.VMEM((8,), jnp.int32),
            pltpu.VMEM((MCAP,), jnp.int32),
            pltpu.VMEM((MCAP,), jnp.int32),
            pltpu.VMEM((CH, D), jnp.float32),
            pltpu.VMEM((CH, D), jnp.float32),
            pltpu.VMEM((512,), jnp.float32),
            pltpu.VMEM((CH,), jnp.float32),
            pltpu.VMEM((CH,), jnp.int32),
            pltpu.VMEM((CH,), jnp.int32),
            pltpu.VMEM((CH,), jnp.int32),
            pltpu.VMEM((CH,), jnp.int32),
            pltpu.VMEM((512,), jnp.float32),
            pltpu.SemaphoreType.DMA,
            pltpu.SemaphoreType.DMA,
            pltpu.SemaphoreType.DMA,
            pltpu.VMEM_SHARED((WROWS, D), jnp.float32),
            pltpu.VMEM_SHARED((WROWS,), jnp.float32),
        ],
        compiler_params=pltpu.CompilerParams(needs_layout_passes=False),
    )
    return f(x, src, comb)


NB = 1000


def _dense_body_l1(bk_ref, dg_ref, x_ref, wc_ref, wr_ref, b_ref, o_ref):
    dg = dg_ref[...]
    nm = jnp.where(dg > 0, 1.0 / jnp.maximum(dg, 1.0), 0.0)
    bk = bk_ref[...] * nm[:, :, None]
    a = bk.reshape(NB, R * D)
    acc = jnp.dot(a, wc_ref[...], preferred_element_type=jnp.float32)
    acc += jnp.dot(x_ref[...], wr_ref[...], preferred_element_type=jnp.float32)
    acc += b_ref[...][None, :]
    o_ref[...] = jnp.maximum(acc, 0.0)


def _dense_body_l2(bk_ref, dg_ref, x_ref, wc_ref, wr_ref, b_ref, rp_ref,
                   z_ref, t2_ref):
    dg = dg_ref[...]
    nm = jnp.where(dg > 0, 1.0 / jnp.maximum(dg, 1.0), 0.0)
    bk = bk_ref[...] * nm[:, :, None]
    a = bk.reshape(NB, R * D)
    acc = jnp.dot(a, wc_ref[...], preferred_element_type=jnp.float32)
    acc += jnp.dot(x_ref[...], wr_ref[...], preferred_element_type=jnp.float32)
    acc += b_ref[...][None, :]
    nrm = jnp.sqrt(jnp.sum(acc * acc, axis=-1, keepdims=True))
    z = acc / jnp.maximum(nrm, 1e-12)
    z_ref[...] = z
    t2_ref[...] = jnp.dot(z, rp_ref[...], preferred_element_type=jnp.float32)


def _dense_layer1(bucket, deg, x, Wcat, W_root, b):
    return pl.pallas_call(
        _dense_body_l1,
        grid=(N // NB,),
        in_specs=[
            pl.BlockSpec((NB, R, D), lambda i: (i, 0, 0)),
            pl.BlockSpec((NB, R), lambda i: (i, 0)),
            pl.BlockSpec((NB, D), lambda i: (i, 0)),
            pl.BlockSpec((R * D, D), lambda i: (0, 0)),
            pl.BlockSpec((D, D), lambda i: (0, 0)),
            pl.BlockSpec((D,), lambda i: (0,)),
        ],
        out_specs=pl.BlockSpec((NB, D), lambda i: (i, 0)),
        out_shape=jax.ShapeDtypeStruct((N, D), jnp.float32),
    )(bucket.reshape(N, R, D), deg.reshape(N, R), x, Wcat, W_root, b)


def _dense_layer2(bucket, deg, x, Wcat, W_root, b, rel_padT):
    return pl.pallas_call(
        _dense_body_l2,
        grid=(N // NB,),
        in_specs=[
            pl.BlockSpec((NB, R, D), lambda i: (i, 0, 0)),
            pl.BlockSpec((NB, R), lambda i: (i, 0)),
            pl.BlockSpec((NB, D), lambda i: (i, 0)),
            pl.BlockSpec((R * D, D), lambda i: (0, 0)),
            pl.BlockSpec((D, D), lambda i: (0, 0)),
            pl.BlockSpec((D,), lambda i: (0,)),
            pl.BlockSpec((D, D), lambda i: (0, 0)),
        ],
        out_specs=[
            pl.BlockSpec((NB, D), lambda i: (i, 0)),
            pl.BlockSpec((NB, D), lambda i: (i, 0)),
        ],
        out_shape=[
            jax.ShapeDtypeStruct((N, D), jnp.float32),
            jax.ShapeDtypeStruct((N, D), jnp.float32),
        ],
    )(bucket.reshape(N, R, D), deg.reshape(N, R), x, Wcat, W_root, b,
      rel_padT)


EPT2 = E // 32
CH2 = 128


def _decode_body(z_hbm, src_hbm, dst_hbm, comb_hbm, t2_hbm, out_hbm,
                 sidx0, didx0, cidx0, sidx1, didx1, cidx1,
                 hst0, tst0, t2b0, ob0, hst1, tst1, t2b1, ob1,
                 isem, gsem, osem):
    c = lax.axis_index("c")
    s = lax.axis_index("s")
    wid = s * NC + c
    NW = NC * NS
    NFULL = (E // NW) // CH2
    NEXTRA = (E - NW * NFULL * CH2) // CH2
    ebase = wid * (NFULL * CH2)
    iota = lax.iota(jnp.int32, 16)

    idxs = ((sidx0, didx0, cidx0), (sidx1, didx1, cidx1))
    rows = ((hst0, tst0, t2b0, ob0), (hst1, tst1, t2b1, ob1))

    def _eoff(j):
        return pl.multiple_of(ebase + j * CH2, 8)

    def _issue_idx(j, p):
        eo = _eoff(j)
        pltpu.async_copy(src_hbm.at[pl.ds(eo, CH2)], idxs[p][0], isem)
        pltpu.async_copy(dst_hbm.at[pl.ds(eo, CH2)], idxs[p][1], isem)
        pltpu.async_copy(comb_hbm.at[pl.ds(eo, CH2)], idxs[p][2], isem)

    def _wait_idx(p):
        for k in range(3):
            pltpu.make_async_copy(src_hbm.at[pl.ds(0, CH2)], idxs[p][k],
                                  isem).wait()

    def _issue_rows(p):
        pltpu.async_copy(z_hbm.at[idxs[p][0]], rows[p][0], gsem)
        pltpu.async_copy(z_hbm.at[idxs[p][1]], rows[p][1], gsem)
        pltpu.async_copy(t2_hbm.at[idxs[p][2]], rows[p][2], gsem)

    def _wait_rows(p):
        pltpu.make_async_copy(z_hbm.at[idxs[p][0]], rows[p][0], gsem).wait()
        pltpu.make_async_copy(z_hbm.at[idxs[p][1]], rows[p][1], gsem).wait()
        pltpu.make_async_copy(t2_hbm.at[idxs[p][2]], rows[p][2], gsem).wait()

    def _compute(j, p):
        hst, tst, t2buf, obuf = rows[p]

        def _edge(e, _):
            acc = hst[e, pl.ds(0, 16)] * tst[e, pl.ds(0, 16)]
            for u in range(1, 8):
                acc += hst[e, pl.ds(u * 16, 16)] * tst[e, pl.ds(u * 16, 16)]
            csum = plsc.cumsum(acc)
            tot = csum.at[jnp.full((16,), 15, jnp.int32)].get(
                mode="promise_in_bounds")
            plsc.store_scatter(obuf, [jnp.full((16,), 0, jnp.int32) + e],
                               tot, mask=iota == 0)
            return 0
        lax.fori_loop(0, CH2, _edge, 0)

        for u in range(8):
            obuf[pl.ds(u * 16, 16)] = (obuf[pl.ds(u * 16, 16)]
                                       + t2buf[pl.ds(u * 16, 16)])
        pltpu.async_copy(obuf, out_hbm.at[pl.ds(_eoff(j), CH2)], osem)

    def _wait_out(p):
        pltpu.make_async_copy(rows[p][3], out_hbm.at[pl.ds(0, CH2)],
                              osem).wait()

    _issue_idx(0, 0)
    _wait_idx(0)
    _issue_rows(0)
    _issue_idx(1, 1)

    def _chunk(j, _):
        for p in (0, 1):
            @pl.when((j & 1) == p)
            def _():
                q = 1 - p
                _wait_rows(p)
                @pl.when(j + 1 < NFULL)
                def _():
                    _wait_idx(q)
                    _issue_rows(q)
                @pl.when(j + 2 < NFULL)
                def _():
                    _issue_idx(j + 2, p)
                @pl.when(j >= 2)
                def _():
                    _wait_out(p)
                _compute(j, p)
        return 0

    lax.fori_loop(0, NFULL, _chunk, 0)
    _wait_out(0)
    _wait_out(1)

    @pl.when(wid < NEXTRA)
    def _():
        eo = pl.multiple_of(NW * NFULL * CH2 + wid * CH2, 8)
        pltpu.sync_copy(src_hbm.at[pl.ds(eo, CH2)], sidx0)
        pltpu.sync_copy(dst_hbm.at[pl.ds(eo, CH2)], didx0)
        pltpu.sync_copy(comb_hbm.at[pl.ds(eo, CH2)], cidx0)
        pltpu.sync_copy(z_hbm.at[sidx0], hst0)
        pltpu.sync_copy(z_hbm.at[didx0], tst0)
        pltpu.sync_copy(t2_hbm.at[cidx0], t2b0)

        def _edge(e, _):
            acc = hst0[e, pl.ds(0, 16)] * tst0[e, pl.ds(0, 16)]
            for u in range(1, 8):
                acc += hst0[e, pl.ds(u * 16, 16)] * tst0[e, pl.ds(u * 16, 16)]
            csum = plsc.cumsum(acc)
            tot = csum.at[jnp.full((16,), 15, jnp.int32)].get(
                mode="promise_in_bounds")
            plsc.store_scatter(ob0, [jnp.full((16,), 0, jnp.int32) + e],
                               tot, mask=iota == 0)
            return 0
        lax.fori_loop(0, CH2, _edge, 0)
        for u in range(8):
            ob0[pl.ds(u * 16, 16)] = (ob0[pl.ds(u * 16, 16)]
                                      + t2b0[pl.ds(u * 16, 16)])
        pltpu.sync_copy(ob0, out_hbm.at[pl.ds(eo, CH2)])


def _decode(z, src, dst, comb, t2):
    mesh = plsc.VectorSubcoreMesh(core_axis_name="c", subcore_axis_name="s")
    f = pl.kernel(
        _decode_body,
        out_type=jax.ShapeDtypeStruct((E,), jnp.float32),
        mesh=mesh,
        scratch_types=[
            pltpu.VMEM((CH2,), jnp.int32),
            pltpu.VMEM((CH2,), jnp.int32),
            pltpu.VMEM((CH2,), jnp.int32),
            pltpu.VMEM((CH2,), jnp.int32),
            pltpu.VMEM((CH2,), jnp.int32),
            pltpu.VMEM((CH2,), jnp.int32),
            pltpu.VMEM((CH2, D), jnp.float32),
            pltpu.VMEM((CH2, D), jnp.float32),
            pltpu.VMEM((CH2,), jnp.float32),
            pltpu.VMEM((CH2,), jnp.float32),
            pltpu.VMEM((CH2, D), jnp.float32),
            pltpu.VMEM((CH2, D), jnp.float32),
            pltpu.VMEM((CH2,), jnp.float32),
            pltpu.VMEM((CH2,), jnp.float32),
            pltpu.SemaphoreType.DMA,
            pltpu.SemaphoreType.DMA,
            pltpu.SemaphoreType.DMA,
        ],
        compiler_params=pltpu.CompilerParams(needs_layout_passes=False),
    )
    return f(z, src, dst, comb, t2)


def kernel(edge_index, edge_type, node_emb, rel_emb, W_root1, W_rel1, b1,
           W_root2, W_rel2, b2):
    src = edge_index[0].astype(jnp.int32)
    dst = edge_index[1].astype(jnp.int32)
    comb = dst * R + edge_type.astype(jnp.int32)

    Wcat1 = W_rel1.reshape(R * D, D)
    Wcat2 = W_rel2.reshape(R * D, D)
    rel_padT = jnp.zeros((D, D), jnp.float32).at[:, :R].set(rel_emb.T)

    bkt1, deg = _bucket(node_emb, src, comb, True)
    x1 = _dense_layer1(bkt1, deg, node_emb, Wcat1, W_root1, b1)
    bkt2 = _bucket(x1, src, comb, False)
    z, t2p = _dense_layer2(bkt2, deg, x1, Wcat2, W_root2, b2, rel_padT)
    t2 = t2p[:, :R].reshape(N * R)
    return _decode(z, src, dst, comb, t2)


# speedup vs baseline: 3.1291x; 3.1291x over previous
import functools

import jax
import jax.numpy as jnp
from jax import lax
from jax.experimental import pallas as pl
from jax.experimental.pallas import tpu as pltpu
from jax.experimental.pallas import tpu_sc as plsc

N = 10000
E = 320000
D = 128
R = 8
NC = 2
NS = 16
C = 8000
WPC = 5
EPT = E // NS
SEG = 4000
NSEG = EPT // SEG
MCAP = SEG + 160
CH = 144
WROWS = 16 * 504


def _make_bucket_body(with_deg):
    def _bucket_body(*args):
        if with_deg:
            (x_hbm, src_hbm, comb_hbm, bucket_hbm, deg_hbm,
             seg_src0, seg_comb0, seg_src1, seg_comb1, m_src, m_comb,
             stag0, stag1, zeros1d, ones1d,
             idx0, sidx0, idx1, sidx1, deg_st, esem, gsem, ssem,
             win, degwin) = args
        else:
            (x_hbm, src_hbm, comb_hbm, bucket_hbm,
             seg_src0, seg_comb0, seg_src1, seg_comb1, m_src, m_comb,
             stag0, stag1, zeros1d, ones1d,
             idx0, sidx0, idx1, sidx1, deg_st, esem, gsem, ssem,
             win, degwin) = args
            deg_hbm = None
        c = lax.axis_index("c")
        s = lax.axis_index("s")

        zf = jnp.zeros((16,), jnp.float32)
        of = jnp.ones((16,), jnp.float32)
        zi = jnp.zeros((16,), jnp.int32)
        cv = jnp.full((16,), C, jnp.int32)
        iota = lax.iota(jnp.int32, 16)

        segs = ((seg_src0, seg_comb0), (seg_src1, seg_comb1))
        stags = (stag0, stag1)
        idxs = ((idx0, sidx0), (idx1, sidx1))

        for u in range(32):
            zeros1d[pl.ds(u * 16, 16)] = zf
        for u in range(9):
            ones1d[pl.ds(u * 16, 16)] = of

        ebase = s * EPT

        def _issue_seg(g, p):
            eoff = pl.multiple_of(ebase + g * SEG, 8)
            pltpu.async_copy(src_hbm.at[pl.ds(eoff, SEG)], segs[p][0], esem)
            pltpu.async_copy(comb_hbm.at[pl.ds(eoff, SEG)], segs[p][1], esem)

        def _wait_seg(p):
            pltpu.make_async_copy(src_hbm.at[pl.ds(0, SEG)], segs[p][0], esem).wait()
            pltpu.make_async_copy(comb_hbm.at[pl.ds(0, SEG)], segs[p][1], esem).wait()

        def _window(w, _):
            wid = c * WPC + w
            base = wid * C
            r0 = s * 504

            def _zs(i, _):
                for u in range(8):
                    stag0[i, pl.ds(u * 16, 16)] = zf
                return 0
            lax.fori_loop(0, 128, _zs, 0)
            pltpu.sync_copy(stag0.at[pl.ds(0, 128)], win.at[pl.ds(r0, 128)])
            pltpu.sync_copy(stag0.at[pl.ds(0, 128)], win.at[pl.ds(r0 + 128, 128)])
            pltpu.sync_copy(stag0.at[pl.ds(0, 128)], win.at[pl.ds(r0 + 256, 128)])
            pltpu.sync_copy(stag0.at[pl.ds(0, 120)], win.at[pl.ds(r0 + 384, 120)])
            if with_deg:
                pltpu.sync_copy(zeros1d.at[pl.ds(0, 504)],
                                degwin.at[pl.ds(r0, 504)])
            plsc.subcore_barrier()

            _issue_seg(0, 0)

            def _process_seg(g, p):
                seg_srcp, seg_combp = segs[p]

                def _scan(i, cntv):
                    vc = seg_combp[pl.ds(i * 16, 16)]
                    vs = seg_srcp[pl.ds(i * 16, 16)]
                    rel = vc - base
                    msk = (rel >= 0) & (rel < C)
                    mi = msk.astype(jnp.int32)
                    pfx = plsc.cumsum(mi)
                    dest = cntv + pfx - 1
                    plsc.store_scatter(m_comb, [dest], rel, mask=msk)
                    plsc.store_scatter(m_src, [dest], vs, mask=msk)
                    return cntv + plsc.all_reduce_population_count(msk)
                cntv = lax.fori_loop(0, SEG // 16, _scan,
                                     jnp.zeros((16,), jnp.int32))

                for u in range(9):
                    pdest = cntv + iota + u * 16
                    plsc.store_scatter(m_comb, [pdest], cv)
                    plsc.store_scatter(m_src, [pdest], zi)

                cnt0 = cntv[0]
                nch = (cnt0 + (CH - 1)) // CH

                def _build_idx(j, q):
                    for u in range(9):
                        gidx = (jnp.full((16,), 0, jnp.int32)
                                + j * CH + u * 16 + iota)
                        idxs[q][0][pl.ds(u * 16, 16)] = plsc.load_gather(
                            m_comb, [gidx])
                        idxs[q][1][pl.ds(u * 16, 16)] = plsc.load_gather(
                            m_src, [gidx])

                def _wait_sca(q):
                    pltpu.make_async_copy(stags[q], win.at[idxs[q][0]],
                                          ssem).wait()
                    if with_deg:
                        pltpu.make_async_copy(ones1d, degwin.at[idxs[q][0]],
                                              ssem).wait()

                @pl.when(nch > 0)
                def _():
                    _build_idx(0, 0)
                    pltpu.async_copy(x_hbm.at[idxs[0][1]], stags[0], gsem)

                def _chunk(j, _):
                    for q in (0, 1):
                        @pl.when((j & 1) == q)
                        def _():
                            pltpu.make_async_copy(x_hbm.at[idxs[q][1]],
                                                  stags[q], gsem).wait()
                            @pl.when(j + 1 < nch)
                            def _():
                                @pl.when(j >= 1)
                                def _():
                                    _wait_sca(1 - q)
                                _build_idx(j + 1, 1 - q)
                                pltpu.async_copy(x_hbm.at[idxs[1 - q][1]],
                                                 stags[1 - q], gsem)
                            pltpu.async_copy(stags[q], win.at[idxs[q][0]],
                                             ssem, add=True)
                            if with_deg:
                                pltpu.async_copy(ones1d,
                                                 degwin.at[idxs[q][0]],
                                                 ssem, add=True)
                    return 0
                lax.fori_loop(0, nch, _chunk, 0)
                for q in (0, 1):
                    @pl.when((nch > 1) & (((nch - 2) & 1) == q))
                    def _():
                        _wait_sca(q)
                    @pl.when((nch > 0) & (((nch - 1) & 1) == q))
                    def _():
                        _wait_sca(q)

            def _segment(g, _):
                for p in (0, 1):
                    @pl.when((g & 1) == p)
                    def _():
                        _wait_seg(p)
                        @pl.when(g + 1 < NSEG)
                        def _():
                            _issue_seg(g + 1, 1 - p)
                        _process_seg(g, p)
                return 0
            lax.fori_loop(0, NSEG, _segment, 0)

            plsc.subcore_barrier()

            @pl.when(s < 15)
            def _():
                pltpu.sync_copy(win.at[pl.ds(r0, 504)],
                                bucket_hbm.at[pl.ds(base + r0, 504)])
                if with_deg:
                    pltpu.sync_copy(degwin.at[pl.ds(r0, 504)],
                                    deg_st.at[pl.ds(0, 504)])
                    pltpu.sync_copy(deg_st.at[pl.ds(0, 504)],
                                    deg_hbm.at[pl.ds(base + r0, 504)])
            @pl.when(s == 15)
            def _():
                pltpu.sync_copy(win.at[pl.ds(15 * 504, 440)],
                                bucket_hbm.at[pl.ds(base + 15 * 504, 440)])
                if with_deg:
                    pltpu.sync_copy(degwin.at[pl.ds(15 * 504, 440)],
                                    deg_st.at[pl.ds(0, 440)])
                    pltpu.sync_copy(deg_st.at[pl.ds(0, 440)],
                                    deg_hbm.at[pl.ds(base + 15 * 504, 440)])
            return 0

        lax.fori_loop(0, WPC, _window, 0)
    return _bucket_body


def _bucket(x, src, comb, with_deg):
    mesh = plsc.VectorSubcoreMesh(core_axis_name="c", subcore_axis_name="s")
    if with_deg:
        out_type = (jax.ShapeDtypeStruct((N * R, D), jnp.float32),
                    jax.ShapeDtypeStruct((N * R,), jnp.float32))
    else:
        out_type = jax.ShapeDtypeStruct((N * R, D), jnp.float32)
    f = pl.kernel(
        _make_bucket_body(with_deg),
        out_type=out_type,
        mesh=mesh,
        scratch_types=[
            pltpu.VMEM((SEG,), jnp.int32),
            pltpu.VMEM((SEG,), jnp.int32),
            pltpu.VMEM((SEG,), jnp.int32),
            pltpu.VMEM((SEG,), jnp.int32),
            pltpu.VMEM((MCAP,), jnp.int32),
            pltpu.VMEM((MCAP,), jnp.int32),
            pltpu.VMEM((CH, D), jnp.float32),
            pltpu.VMEM((CH, D), jnp.float32),
            pltpu.VMEM((512,), jnp.float32),
            pltpu.VMEM((CH,), jnp.float32),
            pltpu.VMEM((CH,), jnp.int32),
            pltpu.VMEM((CH,), jnp.int32),
            pltpu.VMEM((CH,), jnp.int32),
            pltpu.VMEM((CH,), jnp.int32),
            pltpu.VMEM((512,), jnp.float32),
            pltpu.SemaphoreType.DMA,
            pltpu.SemaphoreType.DMA,
            pltpu.SemaphoreType.DMA,
            pltpu.VMEM_SHARED((WROWS, D), jnp.float32),
            pltpu.VMEM_SHARED((WROWS,), jnp.float32),
        ],
        compiler_params=pltpu.CompilerParams(needs_layout_passes=False),
    )
    return f(x, src, comb)


NB = 1000


def _dense_body_l1(bk_ref, dg_ref, x_ref, wc_ref, wr_ref, b_ref, o_ref):
    dg = dg_ref[...]
    nm = jnp.where(dg > 0, 1.0 / jnp.maximum(dg, 1.0), 0.0)
    bk = bk_ref[...] * nm[:, :, None]
    a = bk.reshape(NB, R * D)
    acc = jnp.dot(a, wc_ref[...], preferred_element_type=jnp.float32)
    acc += jnp.dot(x_ref[...], wr_ref[...], preferred_element_type=jnp.float32)
    acc += b_ref[...][None, :]
    o_ref[...] = jnp.maximum(acc, 0.0)


def _dense_body_l2(bk_ref, dg_ref, x_ref, wc_ref, wr_ref, b_ref, rp_ref,
                   z_ref, t2_ref):
    dg = dg_ref[...]
    nm = jnp.where(dg > 0, 1.0 / jnp.maximum(dg, 1.0), 0.0)
    bk = bk_ref[...] * nm[:, :, None]
    a = bk.reshape(NB, R * D)
    acc = jnp.dot(a, wc_ref[...], preferred_element_type=jnp.float32)
    acc += jnp.dot(x_ref[...], wr_ref[...], preferred_element_type=jnp.float32)
    acc += b_ref[...][None, :]
    nrm = jnp.sqrt(jnp.sum(acc * acc, axis=-1, keepdims=True))
    z = acc / jnp.maximum(nrm, 1e-12)
    z_ref[...] = z
    t2_ref[...] = jnp.dot(z, rp_ref[...], preferred_element_type=jnp.float32)


def _dense_layer1(bucket, deg, x, Wcat, W_root, b):
    return pl.pallas_call(
        _dense_body_l1,
        grid=(N // NB,),
        in_specs=[
            pl.BlockSpec((NB, R, D), lambda i: (i, 0, 0)),
            pl.BlockSpec((NB, R), lambda i: (i, 0)),
            pl.BlockSpec((NB, D), lambda i: (i, 0)),
            pl.BlockSpec((R * D, D), lambda i: (0, 0)),
            pl.BlockSpec((D, D), lambda i: (0, 0)),
            pl.BlockSpec((D,), lambda i: (0,)),
        ],
        out_specs=pl.BlockSpec((NB, D), lambda i: (i, 0)),
        out_shape=jax.ShapeDtypeStruct((N, D), jnp.float32),
    )(bucket.reshape(N, R, D), deg.reshape(N, R), x, Wcat, W_root, b)


def _dense_layer2(bucket, deg, x, Wcat, W_root, b, rel_padT):
    return pl.pallas_call(
        _dense_body_l2,
        grid=(N // NB,),
        in_specs=[
            pl.BlockSpec((NB, R, D), lambda i: (i, 0, 0)),
            pl.BlockSpec((NB, R), lambda i: (i, 0)),
            pl.BlockSpec((NB, D), lambda i: (i, 0)),
            pl.BlockSpec((R * D, D), lambda i: (0, 0)),
            pl.BlockSpec((D, D), lambda i: (0, 0)),
            pl.BlockSpec((D,), lambda i: (0,)),
            pl.BlockSpec((D, D), lambda i: (0, 0)),
        ],
        out_specs=[
            pl.BlockSpec((NB, D), lambda i: (i, 0)),
            pl.BlockSpec((NB, D), lambda i: (i, 0)),
        ],
        out_shape=[
            jax.ShapeDtypeStruct((N, D), jnp.float32),
            jax.ShapeDtypeStruct((N, D), jnp.float32),
        ],
    )(bucket.reshape(N, R, D), deg.reshape(N, R), x, Wcat, W_root, b,
      rel_padT)


EPT2 = E // 32
CH2 = 128


def _decode_body(z_hbm, src_hbm, dst_hbm, comb_hbm, t2_hbm, out_hbm,
                 sidx0, didx0, cidx0, sidx1, didx1, cidx1,
                 hst0, tst0, t2b0, ob0, hst1, tst1, t2b1, ob1,
                 isem, gsem, osem):
    c = lax.axis_index("c")
    s = lax.axis_index("s")
    wid = s * NC + c
    NW = NC * NS
    NFULL = (E // NW) // CH2
    NEXTRA = (E - NW * NFULL * CH2) // CH2
    ebase = wid * (NFULL * CH2)
    iota = lax.iota(jnp.int32, 16)

    idxs = ((sidx0, didx0, cidx0), (sidx1, didx1, cidx1))
    rows = ((hst0, tst0, t2b0, ob0), (hst1, tst1, t2b1, ob1))

    def _eoff(j):
        return pl.multiple_of(ebase + j * CH2, 8)

    def _issue_idx(j, p):
        eo = _eoff(j)
        pltpu.async_copy(src_hbm.at[pl.ds(eo, CH2)], idxs[p][0], isem)
        pltpu.async_copy(dst_hbm.at[pl.ds(eo, CH2)], idxs[p][1], isem)
        pltpu.async_copy(comb_hbm.at[pl.ds(eo, CH2)], idxs[p][2], isem)

    def _wait_idx(p):
        for k in range(3):
            pltpu.make_async_copy(src_hbm.at[pl.ds(0, CH2)], idxs[p][k],
                                  isem).wait()

    def _issue_rows(p):
        pltpu.async_copy(z_hbm.at[idxs[p][0]], rows[p][0], gsem)
        pltpu.async_copy(z_hbm.at[idxs[p][1]], rows[p][1], gsem)
        pltpu.async_copy(t2_hbm.at[idxs[p][2]], rows[p][2], gsem)

    def _wait_rows(p):
        pltpu.make_async_copy(z_hbm.at[idxs[p][0]], rows[p][0], gsem).wait()
        pltpu.make_async_copy(z_hbm.at[idxs[p][1]], rows[p][1], gsem).wait()
        pltpu.make_async_copy(t2_hbm.at[idxs[p][2]], rows[p][2], gsem).wait()

    def _compute(j, p):
        hst, tst, t2buf, obuf = rows[p]

        def _edge(e, _):
            acc = hst[e, pl.ds(0, 16)] * tst[e, pl.ds(0, 16)]
            for u in range(1, 8):
                acc += hst[e, pl.ds(u * 16, 16)] * tst[e, pl.ds(u * 16, 16)]
            csum = plsc.cumsum(acc)
            tot = csum.at[jnp.full((16,), 15, jnp.int32)].get(
                mode="promise_in_bounds")
            plsc.store_scatter(obuf, [jnp.full((16,), 0, jnp.int32) + e],
                               tot, mask=iota == 0)
            return 0
        lax.fori_loop(0, CH2, _edge, 0)

        for u in range(8):
            obuf[pl.ds(u * 16, 16)] = (obuf[pl.ds(u * 16, 16)]
                                       + t2buf[pl.ds(u * 16, 16)])
        pltpu.async_copy(obuf, out_hbm.at[pl.ds(_eoff(j), CH2)], osem)

    def _wait_out(p):
        pltpu.make_async_copy(rows[p][3], out_hbm.at[pl.ds(0, CH2)],
                              osem).wait()

    _issue_idx(0, 0)
    _wait_idx(0)
    _issue_rows(0)
    _issue_idx(1, 1)

    def _chunk(j, _):
        for p in (0, 1):
            @pl.when((j & 1) == p)
            def _():
                q = 1 - p
                _wait_rows(p)
                @pl.when(j + 1 < NFULL)
                def _():
                    _wait_idx(q)
                    _issue_rows(q)
                @pl.when(j + 2 < NFULL)
                def _():
                    _issue_idx(j + 2, p)
                @pl.when(j >= 2)
                def _():
                    _wait_out(p)
                _compute(j, p)
        return 0

    lax.fori_loop(0, NFULL, _chunk, 0)
    _wait_out(0)
    _wait_out(1)

    @pl.when(wid < NEXTRA)
    def _():
        eo = pl.multiple_of(NW * NFULL * CH2 + wid * CH2, 8)
        pltpu.sync_copy(src_hbm.at[pl.ds(eo, CH2)], sidx0)
        pltpu.sync_copy(dst_hbm.at[pl.ds(eo, CH2)], didx0)
        pltpu.sync_copy(comb_hbm.at[pl.ds(eo, CH2)], cidx0)
        pltpu.sync_copy(z_hbm.at[sidx0], hst0)
        pltpu.sync_copy(z_hbm.at[didx0], tst0)
        pltpu.sync_copy(t2_hbm.at[cidx0], t2b0)

        def _edge(e, _):
            acc = hst0[e, pl.ds(0, 16)] * tst0[e, pl.ds(0, 16)]
            for u in range(1, 8):
                acc += hst0[e, pl.ds(u * 16, 16)] * tst0[e, pl.ds(u * 16, 16)]
            csum = plsc.cumsum(acc)
            tot = csum.at[jnp.full((16,), 15, jnp.int32)].get(
                mode="promise_in_bounds")
            plsc.store_scatter(ob0, [jnp.full((16,), 0, jnp.int32) + e],
                               tot, mask=iota == 0)
            return 0
        lax.fori_loop(0, CH2, _edge, 0)
        for u in range(8):
            ob0[pl.ds(u * 16, 16)] = (ob0[pl.ds(u * 16, 16)]
                                      + t2b0[pl.ds(u * 16, 16)])
        pltpu.sync_copy(ob0, out_hbm.at[pl.ds(eo, CH2)])


def _decode(z, src, dst, comb, t2):
    mesh = plsc.VectorSubcoreMesh(core_axis_name="c", subcore_axis_name="s")
    f = pl.kernel(
        _decode_body,
        out_type=jax.ShapeDtypeStruct((E,), jnp.float32),
        mesh=mesh,
        scratch_types=[
            pltpu.VMEM((CH2,), jnp.int32),
            pltpu.VMEM((CH2,), jnp.int32),
            pltpu.VMEM((CH2,), jnp.int32),
            pltpu.VMEM((CH2,), jnp.int32),
            pltpu.VMEM((CH2,), jnp.int32),
            pltpu.VMEM((CH2,), jnp.int32),
            pltpu.VMEM((CH2, D), jnp.float32),
            pltpu.VMEM((CH2, D), jnp.float32),
            pltpu.VMEM((CH2,), jnp.float32),
            pltpu.VMEM((CH2,), jnp.float32),
            pltpu.VMEM((CH2, D), jnp.float32),
            pltpu.VMEM((CH2, D), jnp.float32),
            pltpu.VMEM((CH2,), jnp.float32),
            pltpu.VMEM((CH2,), jnp.float32),
            pltpu.SemaphoreType.DMA,
            pltpu.SemaphoreType.DMA,
            pltpu.SemaphoreType.DMA,
        ],
        compiler_params=pltpu.CompilerParams(needs_layout_passes=False),
    )
    return f(z, src, dst, comb, t2)


def kernel(edge_index, edge_type, node_emb, rel_emb, W_root1, W_rel1, b1,
           W_root2, W_rel2, b2):
    src = edge_index[0].astype(jnp.int32)
    dst = edge_index[1].astype(jnp.int32)
    comb = dst * R + edge_type.astype(jnp.int32)

    Wcat1 = W_rel1.reshape(R * D, D)
    Wcat2 = W_rel2.reshape(R * D, D)
    rel_padT = jnp.zeros((D, D), jnp.float32).at[:, :R].set(rel_emb.T)

    bkt1, deg = _bucket(node_emb, src, comb, True)
    x1 = _dense_layer1(bkt1, deg, node_emb, Wcat1, W_root1, b1)
    bkt2 = _bucket(x1, src, comb, False)
    z, t2p = _dense_layer2(bkt2, deg, x1, Wcat2, W_root2, b2, rel_padT)
    t2 = t2p[:, :R].reshape(N * R)
    return _decode(z, src, dst, comb, t2)


# speedup vs baseline: 3.1347x; 1.0018x over previous
import functools

import jax
import jax.numpy as jnp
from jax import lax
from jax.experimental import pallas as pl
from jax.experimental.pallas import tpu as pltpu
from jax.experimental.pallas import tpu_sc as plsc

N = 10000
E = 320000
D = 128
R = 8
NC = 2
NS = 16
C = 8000
WPC = 5
EPT = E // NS
SEG = 4000
NSEG = EPT // SEG
MCAP = SEG + 160
CH = 144
WROWS = 16 * 504


def _make_bucket_body(with_deg):
    def _bucket_body(*args):
        if with_deg:
            (x_hbm, src_hbm, comb_hbm, bucket_hbm, deg_hbm,
             seg_src0, seg_comb0, seg_src1, seg_comb1, m_src, m_comb,
             stag0, stag1, zeros1d, ones1d,
             idx0, sidx0, idx1, sidx1, deg_st, esem, gsem, ssem,
             win, degwin) = args
        else:
            (x_hbm, src_hbm, comb_hbm, bucket_hbm,
             seg_src0, seg_comb0, seg_src1, seg_comb1, m_src, m_comb,
             stag0, stag1, zeros1d, ones1d,
             idx0, sidx0, idx1, sidx1, deg_st, esem, gsem, ssem,
             win, degwin) = args
            deg_hbm = None
        c = lax.axis_index("c")
        s = lax.axis_index("s")

        zf = jnp.zeros((16,), jnp.float32)
        of = jnp.ones((16,), jnp.float32)
        zi = jnp.zeros((16,), jnp.int32)
        cv = jnp.full((16,), C, jnp.int32)
        iota = lax.iota(jnp.int32, 16)

        segs = ((seg_src0, seg_comb0), (seg_src1, seg_comb1))
        stags = (stag0, stag1)
        idxs = ((idx0, sidx0), (idx1, sidx1))

        for u in range(32):
            zeros1d[pl.ds(u * 16, 16)] = zf
        for u in range(9):
            ones1d[pl.ds(u * 16, 16)] = of

        ebase = s * EPT

        def _issue_seg(g, p):
            eoff = pl.multiple_of(ebase + g * SEG, 8)
            pltpu.async_copy(src_hbm.at[pl.ds(eoff, SEG)], segs[p][0], esem)
            pltpu.async_copy(comb_hbm.at[pl.ds(eoff, SEG)], segs[p][1], esem)

        def _wait_seg(p):
            pltpu.make_async_copy(src_hbm.at[pl.ds(0, SEG)], segs[p][0], esem).wait()
            pltpu.make_async_copy(comb_hbm.at[pl.ds(0, SEG)], segs[p][1], esem).wait()

        def _window(w, _):
            wid = c * WPC + w
            base = wid * C
            r0 = s * 504

            def _zs(i, _):
                for u in range(8):
                    stag0[i, pl.ds(u * 16, 16)] = zf
                return 0
            lax.fori_loop(0, 128, _zs, 0)
            pltpu.sync_copy(stag0.at[pl.ds(0, 128)], win.at[pl.ds(r0, 128)])
            pltpu.sync_copy(stag0.at[pl.ds(0, 128)], win.at[pl.ds(r0 + 128, 128)])
            pltpu.sync_copy(stag0.at[pl.ds(0, 128)], win.at[pl.ds(r0 + 256, 128)])
            pltpu.sync_copy(stag0.at[pl.ds(0, 120)], win.at[pl.ds(r0 + 384, 120)])
            if with_deg:
                pltpu.sync_copy(zeros1d.at[pl.ds(0, 504)],
                                degwin.at[pl.ds(r0, 504)])
            plsc.subcore_barrier()

            _issue_seg(0, 0)

            def _process_seg(g, p):
                seg_srcp, seg_combp = segs[p]

                def _scan(i, cntv):
                    vc = seg_combp[pl.ds(i * 16, 16)]
                    vs = seg_srcp[pl.ds(i * 16, 16)]
                    rel = vc - base
                    msk = (rel >= 0) & (rel < C)
                    mi = msk.astype(jnp.int32)
                    pfx = plsc.cumsum(mi)
                    dest = cntv + pfx - 1
                    plsc.store_scatter(m_comb, [dest], rel, mask=msk)
                    plsc.store_scatter(m_src, [dest], vs, mask=msk)
                    return cntv + plsc.all_reduce_population_count(msk)
                cntv = lax.fori_loop(0, SEG // 16, _scan,
                                     jnp.zeros((16,), jnp.int32))

                for u in range(9):
                    pdest = cntv + iota + u * 16
                    plsc.store_scatter(m_comb, [pdest], cv)
                    plsc.store_scatter(m_src, [pdest], zi)

                cnt0 = cntv[0]
                nch = (cnt0 + (CH - 1)) // CH

                def _build_idx(j, q):
                    for u in range(9):
                        gidx = (jnp.full((16,), 0, jnp.int32)
                                + j * CH + u * 16 + iota)
                        idxs[q][0][pl.ds(u * 16, 16)] = plsc.load_gather(
                            m_comb, [gidx])
                        idxs[q][1][pl.ds(u * 16, 16)] = plsc.load_gather(
                            m_src, [gidx])

                def _wait_sca(q):
                    pltpu.make_async_copy(stags[q], win.at[idxs[q][0]],
                                          ssem).wait()
                    if with_deg:
                        pltpu.make_async_copy(ones1d, degwin.at[idxs[q][0]],
                                              ssem).wait()

                @pl.when(nch > 0)
                def _():
                    _build_idx(0, 0)
                    pltpu.async_copy(x_hbm.at[idxs[0][1]], stags[0], gsem)

                def _chunk(j, _):
                    for q in (0, 1):
                        @pl.when((j & 1) == q)
                        def _():
                            pltpu.make_async_copy(x_hbm.at[idxs[q][1]],
                                                  stags[q], gsem).wait()
                            @pl.when(j + 1 < nch)
                            def _():
                                @pl.when(j >= 1)
                                def _():
                                    _wait_sca(1 - q)
                                _build_idx(j + 1, 1 - q)
                                pltpu.async_copy(x_hbm.at[idxs[1 - q][1]],
                                                 stags[1 - q], gsem)
                            pltpu.async_copy(stags[q], win.at[idxs[q][0]],
                                             ssem, add=True)
                            if with_deg:
                                pltpu.async_copy(ones1d,
                                                 degwin.at[idxs[q][0]],
                                                 ssem, add=True)
                    return 0
                lax.fori_loop(0, nch, _chunk, 0)
                for q in (0, 1):
                    @pl.when((nch > 1) & (((nch - 2) & 1) == q))
                    def _():
                        _wait_sca(q)
                    @pl.when((nch > 0) & (((nch - 1) & 1) == q))
                    def _():
                        _wait_sca(q)

            def _segment(g, _):
                for p in (0, 1):
                    @pl.when((g & 1) == p)
                    def _():
                        _wait_seg(p)
                        @pl.when(g + 1 < NSEG)
                        def _():
                            _issue_seg(g + 1, 1 - p)
                        _process_seg(g, p)
                return 0
            lax.fori_loop(0, NSEG, _segment, 0)

            plsc.subcore_barrier()

            @pl.when(s < 15)
            def _():
                pltpu.sync_copy(win.at[pl.ds(r0, 504)],
                                bucket_hbm.at[pl.ds(base + r0, 504)])
                if with_deg:
                    pltpu.sync_copy(degwin.at[pl.ds(r0, 504)],
                                    deg_st.at[pl.ds(0, 504)])
                    pltpu.sync_copy(deg_st.at[pl.ds(0, 504)],
                                    deg_hbm.at[pl.ds(base + r0, 504)])
            @pl.when(s == 15)
            def _():
                pltpu.sync_copy(win.at[pl.ds(15 * 504, 440)],
                                bucket_hbm.at[pl.ds(base + 15 * 504, 440)])
                if with_deg:
                    pltpu.sync_copy(degwin.at[pl.ds(15 * 504, 440)],
                                    deg_st.at[pl.ds(0, 440)])
                    pltpu.sync_copy(deg_st.at[pl.ds(0, 440)],
                                    deg_hbm.at[pl.ds(base + 15 * 504, 440)])
            return 0

        lax.fori_loop(0, WPC, _window, 0)
    return _bucket_body


def _bucket(x, src, comb, with_deg):
    mesh = plsc.VectorSubcoreMesh(core_axis_name="c", subcore_axis_name="s")
    if with_deg:
        out_type = (jax.ShapeDtypeStruct((N * R, D), jnp.float32),
                    jax.ShapeDtypeStruct((N * R,), jnp.float32))
    else:
        out_type = jax.ShapeDtypeStruct((N * R, D), jnp.float32)
    f = pl.kernel(
        _make_bucket_body(with_deg),
        out_type=out_type,
        mesh=mesh,
        scratch_types=[
            pltpu.VMEM((SEG,), jnp.int32),
            pltpu.VMEM((SEG,), jnp.int32),
            pltpu.VMEM((SEG,), jnp.int32),
            pltpu.VMEM((SEG,), jnp.int32),
            pltpu.VMEM((MCAP,), jnp.int32),
            pltpu.VMEM((MCAP,), jnp.int32),
            pltpu.VMEM((CH, D), jnp.float32),
            pltpu.VMEM((CH, D), jnp.float32),
            pltpu.VMEM((512,), jnp.float32),
            pltpu.VMEM((CH,), jnp.float32),
            pltpu.VMEM((CH,), jnp.int32),
            pltpu.VMEM((CH,), jnp.int32),
            pltpu.VMEM((CH,), jnp.int32),
            pltpu.VMEM((CH,), jnp.int32),
            pltpu.VMEM((512,), jnp.float32),
            pltpu.SemaphoreType.DMA,
            pltpu.SemaphoreType.DMA,
            pltpu.SemaphoreType.DMA,
            pltpu.VMEM_SHARED((WROWS, D), jnp.float32),
            pltpu.VMEM_SHARED((WROWS,), jnp.float32),
        ],
        compiler_params=pltpu.CompilerParams(needs_layout_passes=False),
    )
    return f(x, src, comb)


NB = 1000


def _dense_body_l1(bk_ref, dg_ref, x_ref, wc_ref, wr_ref, b_ref, o_ref):
    dg = dg_ref[...]
    nm = jnp.where(dg > 0, 1.0 / jnp.maximum(dg, 1.0), 0.0)
    bk = bk_ref[...] * nm[:, :, None]
    a = bk.reshape(NB, R * D)
    acc = jnp.dot(a, wc_ref[...], preferred_element_type=jnp.float32)
    acc += jnp.dot(x_ref[...], wr_ref[...], preferred_element_type=jnp.float32)
    acc += b_ref[...][None, :]
    o_ref[...] = jnp.maximum(acc, 0.0)


def _dense_body_l2(bk_ref, dg_ref, x_ref, wc_ref, wr_ref, b_ref, rp_ref,
                   z_ref, t2_ref):
    dg = dg_ref[...]
    nm = jnp.where(dg > 0, 1.0 / jnp.maximum(dg, 1.0), 0.0)
    bk = bk_ref[...] * nm[:, :, None]
    a = bk.reshape(NB, R * D)
    acc = jnp.dot(a, wc_ref[...], preferred_element_type=jnp.float32)
    acc += jnp.dot(x_ref[...], wr_ref[...], preferred_element_type=jnp.float32)
    acc += b_ref[...][None, :]
    nrm = jnp.sqrt(jnp.sum(acc * acc, axis=-1, keepdims=True))
    z = acc / jnp.maximum(nrm, 1e-12)
    z_ref[...] = z
    t2_ref[...] = jnp.dot(z, rp_ref[...], preferred_element_type=jnp.float32)


def _dense_layer1(bucket, deg, x, Wcat, W_root, b):
    return pl.pallas_call(
        _dense_body_l1,
        grid=(N // NB,),
        in_specs=[
            pl.BlockSpec((NB, R, D), lambda i: (i, 0, 0)),
            pl.BlockSpec((NB, R), lambda i: (i, 0)),
            pl.BlockSpec((NB, D), lambda i: (i, 0)),
            pl.BlockSpec((R * D, D), lambda i: (0, 0)),
            pl.BlockSpec((D, D), lambda i: (0, 0)),
            pl.BlockSpec((D,), lambda i: (0,)),
        ],
        out_specs=pl.BlockSpec((NB, D), lambda i: (i, 0)),
        out_shape=jax.ShapeDtypeStruct((N, D), jnp.float32),
    )(bucket.reshape(N, R, D), deg.reshape(N, R), x, Wcat, W_root, b)


def _dense_layer2(bucket, deg, x, Wcat, W_root, b, rel_padT):
    return pl.pallas_call(
        _dense_body_l2,
        grid=(N // NB,),
        in_specs=[
            pl.BlockSpec((NB, R, D), lambda i: (i, 0, 0)),
            pl.BlockSpec((NB, R), lambda i: (i, 0)),
            pl.BlockSpec((NB, D), lambda i: (i, 0)),
            pl.BlockSpec((R * D, D), lambda i: (0, 0)),
            pl.BlockSpec((D, D), lambda i: (0, 0)),
            pl.BlockSpec((D,), lambda i: (0,)),
            pl.BlockSpec((D, D), lambda i: (0, 0)),
        ],
        out_specs=[
            pl.BlockSpec((NB, D), lambda i: (i, 0)),
            pl.BlockSpec((NB, D), lambda i: (i, 0)),
        ],
        out_shape=[
            jax.ShapeDtypeStruct((N, D), jnp.float32),
            jax.ShapeDtypeStruct((N, D), jnp.float32),
        ],
    )(bucket.reshape(N, R, D), deg.reshape(N, R), x, Wcat, W_root, b,
      rel_padT)


EPT2 = E // 32
CH2 = 80


def _decode_body(z_hbm, src_hbm, dst_hbm, comb_hbm, t2_hbm, out_hbm,
                 sidx0, didx0, cidx0, sidx1, didx1, cidx1,
                 hst0, tst0, t2b0, ob0, hst1, tst1, t2b1, ob1,
                 isem, gsem, osem):
    c = lax.axis_index("c")
    s = lax.axis_index("s")
    wid = s * NC + c
    NW = NC * NS
    NFULL = (E // NW) // CH2
    NEXTRA = (E - NW * NFULL * CH2) // CH2
    ebase = wid * (NFULL * CH2)
    iota = lax.iota(jnp.int32, 16)

    idxs = ((sidx0, didx0, cidx0), (sidx1, didx1, cidx1))
    rows = ((hst0, tst0, t2b0, ob0), (hst1, tst1, t2b1, ob1))

    def _eoff(j):
        return pl.multiple_of(ebase + j * CH2, 8)

    def _issue_idx(j, p):
        eo = _eoff(j)
        pltpu.async_copy(src_hbm.at[pl.ds(eo, CH2)], idxs[p][0], isem)
        pltpu.async_copy(dst_hbm.at[pl.ds(eo, CH2)], idxs[p][1], isem)
        pltpu.async_copy(comb_hbm.at[pl.ds(eo, CH2)], idxs[p][2], isem)

    def _wait_idx(p):
        for k in range(3):
            pltpu.make_async_copy(src_hbm.at[pl.ds(0, CH2)], idxs[p][k],
                                  isem).wait()

    def _issue_rows(p):
        pltpu.async_copy(z_hbm.at[idxs[p][0]], rows[p][0], gsem)
        pltpu.async_copy(z_hbm.at[idxs[p][1]], rows[p][1], gsem)
        pltpu.async_copy(t2_hbm.at[idxs[p][2]], rows[p][2], gsem)

    def _wait_rows(p):
        pltpu.make_async_copy(z_hbm.at[idxs[p][0]], rows[p][0], gsem).wait()
        pltpu.make_async_copy(z_hbm.at[idxs[p][1]], rows[p][1], gsem).wait()
        pltpu.make_async_copy(t2_hbm.at[idxs[p][2]], rows[p][2], gsem).wait()

    def _compute(j, p):
        hst, tst, t2buf, obuf = rows[p]

        def _edge(e, _):
            acc = hst[e, pl.ds(0, 16)] * tst[e, pl.ds(0, 16)]
            for u in range(1, 8):
                acc += hst[e, pl.ds(u * 16, 16)] * tst[e, pl.ds(u * 16, 16)]
            csum = plsc.cumsum(acc)
            tot = csum.at[jnp.full((16,), 15, jnp.int32)].get(
                mode="promise_in_bounds")
            plsc.store_scatter(obuf, [jnp.full((16,), 0, jnp.int32) + e],
                               tot, mask=iota == 0)
            return 0
        lax.fori_loop(0, CH2, _edge, 0)

        for u in range(5):
            obuf[pl.ds(u * 16, 16)] = (obuf[pl.ds(u * 16, 16)]
                                       + t2buf[pl.ds(u * 16, 16)])
        pltpu.async_copy(obuf, out_hbm.at[pl.ds(_eoff(j), CH2)], osem)

    def _wait_out(p):
        pltpu.make_async_copy(rows[p][3], out_hbm.at[pl.ds(0, CH2)],
                              osem).wait()

    _issue_idx(0, 0)
    _wait_idx(0)
    _issue_rows(0)
    _issue_idx(1, 1)

    def _chunk(j, _):
        for p in (0, 1):
            @pl.when((j & 1) == p)
            def _():
                q = 1 - p
                _wait_rows(p)
                @pl.when(j + 1 < NFULL)
                def _():
                    _wait_idx(q)
                    _issue_rows(q)
                @pl.when(j + 2 < NFULL)
                def _():
                    _issue_idx(j + 2, p)
                @pl.when(j >= 2)
                def _():
                    _wait_out(p)
                _compute(j, p)
        return 0

    lax.fori_loop(0, NFULL, _chunk, 0)
    _wait_out(0)
    _wait_out(1)

    @pl.when(wid < NEXTRA)
    def _():
        eo = pl.multiple_of(NW * NFULL * CH2 + wid * CH2, 8)
        pltpu.sync_copy(src_hbm.at[pl.ds(eo, CH2)], sidx0)
        pltpu.sync_copy(dst_hbm.at[pl.ds(eo, CH2)], didx0)
        pltpu.sync_copy(comb_hbm.at[pl.ds(eo, CH2)], cidx0)
        pltpu.sync_copy(z_hbm.at[sidx0], hst0)
        pltpu.sync_copy(z_hbm.at[didx0], tst0)
        pltpu.sync_copy(t2_hbm.at[cidx0], t2b0)

        def _edge(e, _):
            acc = hst0[e, pl.ds(0, 16)] * tst0[e, pl.ds(0, 16)]
            for u in range(1, 8):
                acc += hst0[e, pl.ds(u * 16, 16)] * tst0[e, pl.ds(u * 16, 16)]
            csum = plsc.cumsum(acc)
            tot = csum.at[jnp.full((16,), 15, jnp.int32)].get(
                mode="promise_in_bounds")
            plsc.store_scatter(ob0, [jnp.full((16,), 0, jnp.int32) + e],
                               tot, mask=iota == 0)
            return 0
        lax.fori_loop(0, CH2, _edge, 0)
        for u in range(5):
            ob0[pl.ds(u * 16, 16)] = (ob0[pl.ds(u * 16, 16)]
                                      + t2b0[pl.ds(u * 16, 16)])
        pltpu.sync_copy(ob0, out_hbm.at[pl.ds(eo, CH2)])


def _decode(z, src, dst, comb, t2):
    mesh = plsc.VectorSubcoreMesh(core_axis_name="c", subcore_axis_name="s")
    f = pl.kernel(
        _decode_body,
        out_type=jax.ShapeDtypeStruct((E,), jnp.float32),
        mesh=mesh,
        scratch_types=[
            pltpu.VMEM((CH2,), jnp.int32),
            pltpu.VMEM((CH2,), jnp.int32),
            pltpu.VMEM((CH2,), jnp.int32),
            pltpu.VMEM((CH2,), jnp.int32),
            pltpu.VMEM((CH2,), jnp.int32),
            pltpu.VMEM((CH2,), jnp.int32),
            pltpu.VMEM((CH2, D), jnp.float32),
            pltpu.VMEM((CH2, D), jnp.float32),
            pltpu.VMEM((CH2,), jnp.float32),
            pltpu.VMEM((CH2,), jnp.float32),
            pltpu.VMEM((CH2, D), jnp.float32),
            pltpu.VMEM((CH2, D), jnp.float32),
            pltpu.VMEM((CH2,), jnp.float32),
            pltpu.VMEM((CH2,), jnp.float32),
            pltpu.SemaphoreType.DMA,
            pltpu.SemaphoreType.DMA,
            pltpu.SemaphoreType.DMA,
        ],
        compiler_params=pltpu.CompilerParams(needs_layout_passes=False),
    )
    return f(z, src, dst, comb, t2)


def kernel(edge_index, edge_type, node_emb, rel_emb, W_root1, W_rel1, b1,
           W_root2, W_rel2, b2):
    src = edge_index[0].astype(jnp.int32)
    dst = edge_index[1].astype(jnp.int32)
    comb = dst * R + edge_type.astype(jnp.int32)

    Wcat1 = W_rel1.reshape(R * D, D)
    Wcat2 = W_rel2.reshape(R * D, D)
    rel_padT = jnp.zeros((D, D), jnp.float32).at[:, :R].set(rel_emb.T)

    bkt1, deg = _bucket(node_emb, src, comb, True)
    x1 = _dense_layer1(bkt1, deg, node_emb, Wcat1, W_root1, b1)
    bkt2 = _bucket(x1, src, comb, False)
    z, t2p = _dense_layer2(bkt2, deg, x1, Wcat2, W_root2, b2, rel_padT)
    t2 = t2p[:, :R].reshape(N * R)
    return _decode(z, src, dst, comb, t2)
